# baseline (speedup 1.0000x reference)
import jax
import jax.numpy as jnp
from jax import lax
from jax.experimental import pallas as pl
from jax.experimental.pallas import tpu as pltpu
from jax.experimental.pallas import tpu_sc as plsc

VOCAB = 1000000
EMB = 32
IMG_F = 512
OUT = 1000
B = 4096
L = 200

NC = 2
NS = 16
NW = NC * NS
ROWS_PER_W = B // NW
R = 8
NCHUNK = ROWS_PER_W // R
CI = R * L
GW = 128
NG = CI // GW


def _cbow_body(words_lo, words_hi, table, h_out, idxa0, idxb0, idxa1, idxb1,
               rows0, rows1, hbuf, sem0, sem1):
    wid = lax.axis_index("s") * NC + lax.axis_index("c")
    base_row = wid * ROWS_PER_W
    idxa_bufs = (idxa0, idxa1)
    idxb_bufs = (idxb0, idxb1)
    rows_bufs = (rows0, rows1)
    sems = (sem0, sem1)

    def fire(c, buf):
        row0 = base_row + c * R
        pltpu.sync_copy(words_lo.at[pl.ds(row0, R), :], idxa_bufs[buf])
        pltpu.sync_copy(words_hi.at[pl.ds(row0, R), :], idxb_bufs[buf])
        for r in range(R):
            pltpu.async_copy(
                table.at[idxa_bufs[buf].at[r, pl.ds(0, GW)]],
                rows_bufs[buf].at[pl.ds(r * L, GW)],
                sems[buf],
            )
            pltpu.async_copy(
                table.at[idxb_bufs[buf].at[r, pl.ds(0, L - GW)]],
                rows_bufs[buf].at[pl.ds(r * L + GW, L - GW)],
                sems[buf],
            )

    def drain(buf):
        pltpu.make_async_copy(
            table.at[pl.ds(0, CI)], rows_bufs[buf], sems[buf]
        ).wait()

    def process(c, buf):
        rows_v = rows_bufs[buf]
        for r in range(R):
            zf = jnp.zeros((16,), jnp.float32)

            def body(i, accs):
                a0, a1, a2, a3 = accs
                row = r * L + 2 * i
                a0 = a0 + rows_v[row, 0:16]
                a1 = a1 + rows_v[row, 16:32]
                a2 = a2 + rows_v[row + 1, 0:16]
                a3 = a3 + rows_v[row + 1, 16:32]
                return (a0, a1, a2, a3)

            a0, a1, a2, a3 = lax.fori_loop(0, L // 2, body, (zf, zf, zf, zf))
            rloc = c * R + r
            hbuf[rloc, 0:16] = a0 + a2
            hbuf[rloc, 16:32] = a1 + a3

    fire(0, 0)

    def outer(cc, carry):
        c0 = 2 * cc
        fire(c0 + 1, 1)
        drain(0)
        process(c0, 0)
        fire(c0 + 2, 0)
        drain(1)
        process(c0 + 1, 1)
        return carry

    lax.fori_loop(0, NCHUNK // 2 - 1, outer, 0)
    c0 = NCHUNK - 2
    fire(c0 + 1, 1)
    drain(0)
    process(c0, 0)
    drain(1)
    process(c0 + 1, 1)

    pltpu.sync_copy(hbuf, h_out.at[pl.ds(wid * ROWS_PER_W, ROWS_PER_W)])


def _cbow_sum(words, table):
    words_lo = words[:, :GW]
    words_hi = jnp.pad(words[:, GW:], ((0, 0), (0, 2 * GW - L)))
    mesh = plsc.VectorSubcoreMesh(core_axis_name="c", subcore_axis_name="s")
    kern = pl.kernel(
        _cbow_body,
        out_type=jax.ShapeDtypeStruct((B, EMB), jnp.float32),
        mesh=mesh,
        scratch_types=[
            pltpu.VMEM((R, GW), jnp.int32),
            pltpu.VMEM((R, GW), jnp.int32),
            pltpu.VMEM((R, GW), jnp.int32),
            pltpu.VMEM((R, GW), jnp.int32),
            pltpu.VMEM((CI, EMB), jnp.float32),
            pltpu.VMEM((CI, EMB), jnp.float32),
            pltpu.VMEM((ROWS_PER_W, EMB), jnp.float32),
            pltpu.SemaphoreType.DMA,
            pltpu.SemaphoreType.DMA,
        ],
        compiler_params=pltpu.CompilerParams(use_tc_tiling_on_sc=False),
    )
    return kern(words_lo, words_hi, table)


def _mlp_body(image_ref, h_ref, words_ref, t0_ref, w1_ref, b1_ref, out_ref):
    x = image_ref[...]
    cnt = jnp.sum((words_ref[...] == 0).astype(jnp.float32), axis=1,
                  keepdims=True)
    h = h_ref[...] - cnt * t0_ref[0:1, :]
    wa = w1_ref[:, 0:IMG_F]
    wb = w1_ref[:, IMG_F:IMG_F + EMB]
    acc = lax.dot_general(x, wa, (((1,), (1,)), ((), ())),
                          preferred_element_type=jnp.float32)
    acc = acc + lax.dot_general(h, wb, (((1,), (1,)), ((), ())),
                                preferred_element_type=jnp.float32)
    out_ref[...] = acc + b1_ref[...][None, :]


def _mlp(image, h, words, t0, W1, b1):
    BLK = 1024
    grid = (B // BLK,)
    return pl.pallas_call(
        _mlp_body,
        grid=grid,
        in_specs=[
            pl.BlockSpec((BLK, IMG_F), lambda i: (i, 0)),
            pl.BlockSpec((BLK, EMB), lambda i: (i, 0)),
            pl.BlockSpec((BLK, L), lambda i: (i, 0)),
            pl.BlockSpec((8, EMB), lambda i: (0, 0)),
            pl.BlockSpec((OUT, IMG_F + EMB), lambda i: (0, 0)),
            pl.BlockSpec((OUT,), lambda i: (0,)),
        ],
        out_specs=pl.BlockSpec((BLK, OUT), lambda i: (i, 0)),
        out_shape=jax.ShapeDtypeStruct((B, OUT), jnp.float32),
    )(image, h, words, t0, W1, b1)


@jax.jit
def kernel(words, image, table, W1, b1):
    words = words.astype(jnp.int32)
    h = _cbow_sum(words, table)
    return _mlp(image, h, words, table[0:8], W1, b1)
